# baseline (speedup 1.0000x reference)
import functools

import jax
import jax.numpy as jnp
from jax import lax
from jax.experimental import pallas as pl
from jax.experimental.pallas import tpu as pltpu
from jax.experimental.pallas import tpu_sc as plsc

N_NODES = 10000
N_EDGES = 320000
D = 128
DE = 16
DT = 256

NC, NS = 2, 16
NW = NC * NS
EPW = N_EDGES // NW
GC = 80
NCHUNK = EPW // GC
NB = N_NODES // GC
NBT = (NB + NS - 1) // NS


def _pack_bf16_pair(lo_f32, hi_f32):
    half = jnp.uint32(0x8000)
    lo = (lax.bitcast_convert_type(lo_f32, jnp.uint32) + half) >> 16
    hi = (lax.bitcast_convert_type(hi_f32, jnp.uint32) + half) & jnp.uint32(0xFFFF0000)
    return lax.bitcast_convert_type(lo | hi, jnp.float32)


def _tables_body(x_ref, wa_ref, wb_ref, ta_ref, tb_ref):
    x = x_ref[...]
    ta = jnp.dot(x, wa_ref[...], preferred_element_type=jnp.float32)
    tb = jnp.dot(x, wb_ref[...], preferred_element_type=jnp.float32)
    ta_ref[...] = _pack_bf16_pair(ta[:, :D], ta[:, D:])
    tb_ref[...] = _pack_bf16_pair(tb[:, :D], tb[:, D:])


def _tables(x, wa, wb):
    blk = 2000
    return pl.pallas_call(
        _tables_body,
        grid=(N_NODES // blk,),
        in_specs=[
            pl.BlockSpec((blk, D), lambda i: (i, 0)),
            pl.BlockSpec((D, DT), lambda i: (0, 0)),
            pl.BlockSpec((D, DT), lambda i: (0, 0)),
        ],
        out_specs=[
            pl.BlockSpec((blk, D), lambda i: (i, 0)),
            pl.BlockSpec((blk, D), lambda i: (i, 0)),
        ],
        out_shape=[jax.ShapeDtypeStruct((N_NODES, D), jnp.float32)] * 2,
    )(x, wa, wb)


@functools.partial(
    pl.kernel,
    out_type=(jax.ShapeDtypeStruct((N_EDGES, D), jnp.float32),
              jax.ShapeDtypeStruct((N_EDGES, D), jnp.float32)),
    mesh=plsc.VectorSubcoreMesh(core_axis_name="c", subcore_axis_name="s"),
    scratch_types=[
        pltpu.VMEM((2, GC), jnp.int32),
        pltpu.VMEM((2, GC), jnp.int32),
        pltpu.VMEM((2, GC, D), jnp.float32),
        pltpu.VMEM((2, GC, D), jnp.float32),
        [pltpu.SemaphoreType.DMA] * 2,
        [pltpu.SemaphoreType.DMA] * 2,
        [pltpu.SemaphoreType.DMA] * 2,
        [pltpu.SemaphoreType.DMA] * 2,
    ],
)
def _sc_gather(ta, tb, src, tgt, g1, g2, ia, ib, ba, bb,
               sem_a, sem_b, sem_wa, sem_wb):
    wid = lax.axis_index("c") * NS + lax.axis_index("s")
    ebase = wid * EPW

    def fire(k, b):
        pltpu.sync_copy(src.at[pl.ds(ebase + k * GC, GC)], ia.at[b])
        pltpu.sync_copy(tgt.at[pl.ds(ebase + k * GC, GC)], ib.at[b])
        pltpu.async_copy(ta.at[ia.at[b]], ba.at[b], sem_a[b])
        pltpu.async_copy(tb.at[ib.at[b]], bb.at[b], sem_b[b])

    def drain_gather(b):
        pltpu.make_async_copy(ta.at[ia.at[b]], ba.at[b], sem_a[b]).wait()
        pltpu.make_async_copy(tb.at[ib.at[b]], bb.at[b], sem_b[b]).wait()

    def fire_write(k, b):
        pltpu.async_copy(ba.at[b], g1.at[pl.ds(ebase + k * GC, GC)], sem_wa[b])
        pltpu.async_copy(bb.at[b], g2.at[pl.ds(ebase + k * GC, GC)], sem_wb[b])

    def drain_write(b):
        pltpu.make_async_copy(ba.at[b], g1.at[pl.ds(ebase, GC)], sem_wa[b]).wait()
        pltpu.make_async_copy(bb.at[b], g2.at[pl.ds(ebase, GC)], sem_wb[b]).wait()

    fire(0, 0)

    def pair(p, carry):
        k0 = p * 2

        @pl.when(p > 0)
        def _():
            drain_write(1)

        fire(k0 + 1, 1)
        drain_gather(0)
        fire_write(k0, 0)

        @pl.when(k0 + 2 < NCHUNK)
        def _():
            drain_write(0)
            fire(k0 + 2, 0)

        drain_gather(1)
        fire_write(k0 + 1, 1)
        return carry

    lax.fori_loop(0, NCHUNK // 2, pair, 0)
    drain_gather(0)
    fire_write(NCHUNK - 1, 0)
    drain_write(0)
    drain_write(1)


def _unpack_bf16_pair(gw):
    zf = lax.bitcast_convert_type(gw << 16, jnp.float32)
    zs = lax.bitcast_convert_type(gw & jnp.uint32(0xFFFF0000), jnp.float32)
    return zf, zs


def _msg_body(g1_ref, g2_ref, e_ref, we_ref, b_ref, m_ref):
    f1, s1 = _unpack_bf16_pair(lax.bitcast_convert_type(g1_ref[...], jnp.uint32))
    f2, s2 = _unpack_bf16_pair(lax.bitcast_convert_type(g2_ref[...], jnp.uint32))
    ez = jnp.dot(e_ref[...], we_ref[...],
                 preferred_element_type=jnp.float32) + b_ref[...]
    zf = f1 + f2 + ez[:, :D]
    zs = s1 + s2 + ez[:, D:]
    sig = 1.0 / (1.0 + jnp.exp(-zf))
    sp = jnp.maximum(zs, 0.0) + jnp.log(1.0 + jnp.exp(-jnp.abs(zs)))
    m_ref[...] = sig * sp


def _msg(g1, g2, ea, we, bcat):
    blk = 12800
    return pl.pallas_call(
        _msg_body,
        grid=(N_EDGES // blk,),
        in_specs=[
            pl.BlockSpec((blk, D), lambda i: (i, 0)),
            pl.BlockSpec((blk, D), lambda i: (i, 0)),
            pl.BlockSpec((blk, DE), lambda i: (i, 0)),
            pl.BlockSpec((DE, DT), lambda i: (0, 0)),
            pl.BlockSpec((1, DT), lambda i: (0, 0)),
        ],
        out_specs=pl.BlockSpec((blk, D), lambda i: (i, 0)),
        out_shape=jax.ShapeDtypeStruct((N_EDGES, D), jnp.float32),
    )(g1, g2, ea, we, bcat)


@functools.partial(
    pl.kernel,
    out_type=jax.ShapeDtypeStruct((NC, N_NODES, D), jnp.float32),
    mesh=plsc.VectorSubcoreMesh(core_axis_name="c", subcore_axis_name="s"),
    scratch_types=[
        pltpu.VMEM((2, GC), jnp.int32),
        pltpu.VMEM((2, GC, D), jnp.float32),
        pltpu.VMEM_SHARED((N_NODES, D), jnp.float32),
        [pltpu.SemaphoreType.DMA] * 2,
    ],
)
def _sc_scatter(msg, src, out, idx, buf, acc, sem):
    c = lax.axis_index("c")
    s = lax.axis_index("s")
    ebase = (c * NS + s) * EPW
    zeros = jnp.zeros((16,), jnp.float32)

    def zrow(r, carry):
        for j in range(D // 16):
            buf[0, r, pl.ds(j * 16, 16)] = zeros
        return carry

    lax.fori_loop(0, GC, zrow, 0)

    def initblk(t, carry):
        b = s + t * NS
        @pl.when(b < NB)
        def _():
            pltpu.sync_copy(buf.at[0], acc.at[pl.ds(b * GC, GC)])
        return carry

    lax.fori_loop(0, NBT, initblk, 0)
    plsc.subcore_barrier()

    def fire(k, b):
        pltpu.sync_copy(src.at[pl.ds(ebase + k * GC, GC)], idx.at[b])
        pltpu.async_copy(msg.at[pl.ds(ebase + k * GC, GC)], buf.at[b], sem[b])

    def scat(b):
        pltpu.make_async_copy(msg.at[pl.ds(ebase, GC)], buf.at[b], sem[b]).wait()
        pltpu.sync_copy(buf.at[b], acc.at[idx.at[b]], add=True)

    fire(0, 0)

    def pair(p, carry):
        k0 = p * 2
        fire(k0 + 1, 1)
        scat(0)

        @pl.when(k0 + 2 < NCHUNK)
        def _():
            fire(k0 + 2, 0)

        scat(1)
        return carry

    lax.fori_loop(0, NCHUNK // 2, pair, 0)
    scat(0)
    plsc.subcore_barrier()

    def outblk(t, carry):
        b = s + t * NS
        @pl.when(b < NB)
        def _():
            pltpu.sync_copy(acc.at[pl.ds(b * GC, GC)], buf.at[0])
            pltpu.sync_copy(buf.at[0], out.at[c, pl.ds(b * GC, GC)])
        return carry

    lax.fori_loop(0, NBT, outblk, 0)


def _final_body(p_ref, x_ref, g_ref, b_ref, o_ref):
    m = p_ref[0] + p_ref[1]
    mean = jnp.mean(m, axis=0, keepdims=True)
    cent = m - mean
    var = jnp.mean(cent * cent, axis=0, keepdims=True)
    o_ref[...] = x_ref[...] + cent * lax.rsqrt(var + 1e-5) * g_ref[...] + b_ref[...]


def _final(parts, x, gamma, beta):
    return pl.pallas_call(
        _final_body,
        out_shape=jax.ShapeDtypeStruct((N_NODES, D), jnp.float32),
    )(parts, x, gamma, beta)


def kernel(node_attrs, edge_index, edge_attrs, Wf, bf, Ws, bs, gamma, beta):
    src = edge_index[0].astype(jnp.int32)
    tgt = edge_index[1].astype(jnp.int32)
    wa = jnp.concatenate([Wf[:, :D], Ws[:, :D]], axis=0).T
    wb = jnp.concatenate([Wf[:, D:2 * D], Ws[:, D:2 * D]], axis=0).T
    we = jnp.concatenate([Wf[:, 2 * D:], Ws[:, 2 * D:]], axis=0).T
    bcat = jnp.concatenate([bf, bs])[None, :]

    ta, tb = _tables(node_attrs, wa, wb)
    g1, g2 = _sc_gather(ta, tb, src, tgt)
    m = _msg(g1, g2, edge_attrs, we, bcat)
    parts = _sc_scatter(m, src)
    return _final(parts, node_attrs, gamma[None, :], beta[None, :])


# speedup vs baseline: 1.0066x; 1.0066x over previous
import functools

import jax
import jax.numpy as jnp
from jax import lax
from jax.experimental import pallas as pl
from jax.experimental.pallas import tpu as pltpu
from jax.experimental.pallas import tpu_sc as plsc

N_NODES = 10000
N_EDGES = 320000
D = 128
DE = 16
DT = 256

NC, NS = 2, 16
NW = NC * NS
EPW = N_EDGES // NW
GC = 80
NCHUNK = EPW // GC
NB = N_NODES // GC
NBT = (NB + NS - 1) // NS


def _pack_bf16_pair(lo_f32, hi_f32):
    half = jnp.uint32(0x8000)
    lo = (lax.bitcast_convert_type(lo_f32, jnp.uint32) + half) >> 16
    hi = (lax.bitcast_convert_type(hi_f32, jnp.uint32) + half) & jnp.uint32(0xFFFF0000)
    return lax.bitcast_convert_type(lo | hi, jnp.float32)


def _tables_body(x_ref, wa_ref, wb_ref, ta_ref, tb_ref):
    x = x_ref[...]
    ta = jnp.dot(x, wa_ref[...], preferred_element_type=jnp.float32)
    tb = jnp.dot(x, wb_ref[...], preferred_element_type=jnp.float32)
    ta_ref[...] = _pack_bf16_pair(ta[:, :D], ta[:, D:])
    tb_ref[...] = _pack_bf16_pair(tb[:, :D], tb[:, D:])


def _tables(x, wa, wb):
    blk = 2000
    return pl.pallas_call(
        _tables_body,
        grid=(N_NODES // blk,),
        in_specs=[
            pl.BlockSpec((blk, D), lambda i: (i, 0)),
            pl.BlockSpec((D, DT), lambda i: (0, 0)),
            pl.BlockSpec((D, DT), lambda i: (0, 0)),
        ],
        out_specs=[
            pl.BlockSpec((blk, D), lambda i: (i, 0)),
            pl.BlockSpec((blk, D), lambda i: (i, 0)),
        ],
        out_shape=[jax.ShapeDtypeStruct((N_NODES, D), jnp.float32)] * 2,
    )(x, wa, wb)


@functools.partial(
    pl.kernel,
    out_type=(jax.ShapeDtypeStruct((N_EDGES, D), jnp.float32),
              jax.ShapeDtypeStruct((N_EDGES, D), jnp.float32)),
    mesh=plsc.VectorSubcoreMesh(core_axis_name="c", subcore_axis_name="s"),
    scratch_types=[
        pltpu.VMEM((2, GC), jnp.int32),
        pltpu.VMEM((2, GC), jnp.int32),
        pltpu.VMEM((2, GC, D), jnp.float32),
        pltpu.VMEM((2, GC, D), jnp.float32),
        [pltpu.SemaphoreType.DMA] * 2,
        [pltpu.SemaphoreType.DMA] * 2,
        [pltpu.SemaphoreType.DMA] * 2,
        [pltpu.SemaphoreType.DMA] * 2,
    ],
)
def _sc_gather(ta, tb, src, tgt, g1, g2, ia, ib, ba, bb,
               sem_a, sem_b, sem_wa, sem_wb):
    wid = lax.axis_index("c") * NS + lax.axis_index("s")
    ebase = wid * EPW

    def fire(k, b):
        pltpu.sync_copy(src.at[pl.ds(ebase + k * GC, GC)], ia.at[b])
        pltpu.sync_copy(tgt.at[pl.ds(ebase + k * GC, GC)], ib.at[b])
        pltpu.async_copy(ta.at[ia.at[b]], ba.at[b], sem_a[b])
        pltpu.async_copy(tb.at[ib.at[b]], bb.at[b], sem_b[b])

    def drain_gather(b):
        pltpu.make_async_copy(ta.at[ia.at[b]], ba.at[b], sem_a[b]).wait()
        pltpu.make_async_copy(tb.at[ib.at[b]], bb.at[b], sem_b[b]).wait()

    def fire_write(k, b):
        pltpu.async_copy(ba.at[b], g1.at[pl.ds(ebase + k * GC, GC)], sem_wa[b])
        pltpu.async_copy(bb.at[b], g2.at[pl.ds(ebase + k * GC, GC)], sem_wb[b])

    def drain_write(b):
        pltpu.make_async_copy(ba.at[b], g1.at[pl.ds(ebase, GC)], sem_wa[b]).wait()
        pltpu.make_async_copy(bb.at[b], g2.at[pl.ds(ebase, GC)], sem_wb[b]).wait()

    fire(0, 0)

    def pair(p, carry):
        k0 = p * 2

        @pl.when(p > 0)
        def _():
            drain_write(1)

        fire(k0 + 1, 1)
        drain_gather(0)
        fire_write(k0, 0)

        @pl.when(k0 + 2 < NCHUNK)
        def _():
            drain_write(0)
            fire(k0 + 2, 0)

        drain_gather(1)
        fire_write(k0 + 1, 1)
        return carry

    lax.fori_loop(0, NCHUNK // 2, pair, 0)
    drain_gather(0)
    fire_write(NCHUNK - 1, 0)
    drain_write(0)
    drain_write(1)


def _unpack_bf16_pair(gw):
    zf = lax.bitcast_convert_type(gw << 16, jnp.float32)
    zs = lax.bitcast_convert_type(gw & jnp.uint32(0xFFFF0000), jnp.float32)
    return zf, zs


def _msg_body(g1_ref, g2_ref, e_ref, we_ref, b_ref, m_ref):
    f1, s1 = _unpack_bf16_pair(lax.bitcast_convert_type(g1_ref[...], jnp.uint32))
    f2, s2 = _unpack_bf16_pair(lax.bitcast_convert_type(g2_ref[...], jnp.uint32))
    ez = jnp.dot(e_ref[...], we_ref[...],
                 preferred_element_type=jnp.float32) + b_ref[...]
    zf = f1 + f2 + ez[:, :D]
    zs = s1 + s2 + ez[:, D:]
    sig = 1.0 / (1.0 + jnp.exp(-zf))
    sp = jnp.maximum(zs, 0.0) + jnp.log(1.0 + jnp.exp(-jnp.abs(zs)))
    m_ref[...] = sig * sp


def _msg(g1, g2, ea, we, bcat):
    blk = 6400
    return pl.pallas_call(
        _msg_body,
        grid=(N_EDGES // blk,),
        in_specs=[
            pl.BlockSpec((blk, D), lambda i: (i, 0)),
            pl.BlockSpec((blk, D), lambda i: (i, 0)),
            pl.BlockSpec((blk, DE), lambda i: (i, 0)),
            pl.BlockSpec((DE, DT), lambda i: (0, 0)),
            pl.BlockSpec((1, DT), lambda i: (0, 0)),
        ],
        out_specs=pl.BlockSpec((blk, D), lambda i: (i, 0)),
        out_shape=jax.ShapeDtypeStruct((N_EDGES, D), jnp.float32),
    )(g1, g2, ea, we, bcat)


@functools.partial(
    pl.kernel,
    out_type=jax.ShapeDtypeStruct((NC, N_NODES, D), jnp.float32),
    mesh=plsc.VectorSubcoreMesh(core_axis_name="c", subcore_axis_name="s"),
    scratch_types=[
        pltpu.VMEM((2, GC), jnp.int32),
        pltpu.VMEM((2, GC, D), jnp.float32),
        pltpu.VMEM_SHARED((N_NODES, D), jnp.float32),
        [pltpu.SemaphoreType.DMA] * 2,
    ],
)
def _sc_scatter(msg, src, out, idx, buf, acc, sem):
    c = lax.axis_index("c")
    s = lax.axis_index("s")
    ebase = (c * NS + s) * EPW
    zeros = jnp.zeros((16,), jnp.float32)

    def zrow(r, carry):
        for j in range(D // 16):
            buf[0, r, pl.ds(j * 16, 16)] = zeros
        return carry

    lax.fori_loop(0, GC, zrow, 0)

    def initblk(t, carry):
        b = s + t * NS
        @pl.when(b < NB)
        def _():
            pltpu.sync_copy(buf.at[0], acc.at[pl.ds(b * GC, GC)])
        return carry

    lax.fori_loop(0, NBT, initblk, 0)
    plsc.subcore_barrier()

    def fire(k, b):
        pltpu.sync_copy(src.at[pl.ds(ebase + k * GC, GC)], idx.at[b])
        pltpu.async_copy(msg.at[pl.ds(ebase + k * GC, GC)], buf.at[b], sem[b])

    def scat(b):
        pltpu.make_async_copy(msg.at[pl.ds(ebase, GC)], buf.at[b], sem[b]).wait()
        pltpu.sync_copy(buf.at[b], acc.at[idx.at[b]], add=True)

    fire(0, 0)

    def pair(p, carry):
        k0 = p * 2
        fire(k0 + 1, 1)
        scat(0)

        @pl.when(k0 + 2 < NCHUNK)
        def _():
            fire(k0 + 2, 0)

        scat(1)
        return carry

    lax.fori_loop(0, NCHUNK // 2, pair, 0)
    scat(0)
    plsc.subcore_barrier()

    def outblk(t, carry):
        b = s + t * NS
        @pl.when(b < NB)
        def _():
            pltpu.sync_copy(acc.at[pl.ds(b * GC, GC)], buf.at[0])
            pltpu.sync_copy(buf.at[0], out.at[c, pl.ds(b * GC, GC)])
        return carry

    lax.fori_loop(0, NBT, outblk, 0)


def _final_body(p_ref, x_ref, g_ref, b_ref, o_ref):
    m = p_ref[0] + p_ref[1]
    mean = jnp.mean(m, axis=0, keepdims=True)
    cent = m - mean
    var = jnp.mean(cent * cent, axis=0, keepdims=True)
    o_ref[...] = x_ref[...] + cent * lax.rsqrt(var + 1e-5) * g_ref[...] + b_ref[...]


def _final(parts, x, gamma, beta):
    return pl.pallas_call(
        _final_body,
        out_shape=jax.ShapeDtypeStruct((N_NODES, D), jnp.float32),
    )(parts, x, gamma, beta)


def kernel(node_attrs, edge_index, edge_attrs, Wf, bf, Ws, bs, gamma, beta):
    src = edge_index[0].astype(jnp.int32)
    tgt = edge_index[1].astype(jnp.int32)
    wa = jnp.concatenate([Wf[:, :D], Ws[:, :D]], axis=0).T
    wb = jnp.concatenate([Wf[:, D:2 * D], Ws[:, D:2 * D]], axis=0).T
    we = jnp.concatenate([Wf[:, 2 * D:], Ws[:, 2 * D:]], axis=0).T
    bcat = jnp.concatenate([bf, bs])[None, :]

    ta, tb = _tables(node_attrs, wa, wb)
    g1, g2 = _sc_gather(ta, tb, src, tgt)
    m = _msg(g1, g2, edge_attrs, we, bcat)
    parts = _sc_scatter(m, src)
    return _final(parts, node_attrs, gamma[None, :], beta[None, :])
